# baseline (speedup 1.0000x reference)
import functools

import jax
import jax.numpy as jnp
from jax import lax
from jax.experimental import pallas as pl
from jax.experimental.pallas import tpu as pltpu
from jax.experimental.pallas import tpu_sc as plsc

H = 512
W = 512
NPH = 32
NPW = 32
PH = H // NPH
PW = W // NPW
NP = NPH * NPW
C = 192

NC = 2
NS = 16
NW = NC * NS

ROWBYTES = PW * C
WCOLS = NPW * ROWBYTES


def _patch_body(x_hbm, out_hbm, sem):
    wid = lax.axis_index("s") * NC + lax.axis_index("c")

    def step(j, _):
        pltpu.async_copy(
            x_hbm.at[pl.ds(wid * PH, PH), pl.ds(j * ROWBYTES, ROWBYTES)],
            out_hbm.at[pl.ds(wid * NPW * PH + j * PH, PH), :],
            sem,
        )
        return 0

    lax.fori_loop(0, NPW, step, 0)

    def drain(j, _):
        pltpu.make_async_copy(
            x_hbm.at[pl.ds(0, PH), pl.ds(0, ROWBYTES)],
            out_hbm.at[pl.ds(0, PH), :],
            sem,
        ).wait()
        return 0

    lax.fori_loop(0, NPW, drain, 0)


_patch_kernel = functools.partial(
    pl.kernel,
    out_type=jax.ShapeDtypeStruct((NP * PH, ROWBYTES), jnp.float32),
    mesh=plsc.VectorSubcoreMesh(
        core_axis_name="c", subcore_axis_name="s", num_cores=NC, num_subcores=NS
    ),
    scratch_types=[pltpu.SemaphoreType.DMA],
)(_patch_body)


def kernel(x, mesh_pos, batch_idx):
    x2 = x.reshape(H, WCOLS)
    out = _patch_kernel(x2)
    return out.reshape(1, NP, PH, PW, C)


# speedup vs baseline: 7.3488x; 7.3488x over previous
import functools

import jax
import jax.numpy as jnp
from jax import lax
from jax.experimental import pallas as pl
from jax.experimental.pallas import tpu as pltpu
from jax.experimental.pallas import tpu_sc as plsc

H = 512
W = 512
NPH = 32
NPW = 32
PH = H // NPH
PW = W // NPW
NP = NPH * NPW
C = 192

NC = 2
NS = 16
NW = NC * NS

ROWBYTES = PW * C
WCOLS = NPW * ROWBYTES


def _patch_body(x_hbm, out_hbm, buf0, buf1, sl0, sl1, ss0, ss1):
    wid = lax.axis_index("s") * NC + lax.axis_index("c")
    row0 = wid * PH
    obase = wid * NPW * PH

    def src(j):
        return x_hbm.at[pl.ds(row0, PH), pl.ds(j * ROWBYTES, ROWBYTES)]

    def dst(j):
        return out_hbm.at[pl.ds(obase + j * PH, PH), :]

    pltpu.async_copy(src(0), buf0, sl0)
    pltpu.async_copy(src(1), buf1, sl1)

    def step(i, _):
        j = 2 * i
        pltpu.make_async_copy(src(j), buf0, sl0).wait()
        pltpu.async_copy(buf0, dst(j), ss0)
        pltpu.make_async_copy(src(j + 1), buf1, sl1).wait()
        pltpu.async_copy(buf1, dst(j + 1), ss1)
        pltpu.make_async_copy(buf0, dst(j), ss0).wait()
        pltpu.async_copy(src(j + 2), buf0, sl0)
        pltpu.make_async_copy(buf1, dst(j + 1), ss1).wait()
        pltpu.async_copy(src(j + 3), buf1, sl1)
        return 0

    lax.fori_loop(0, (NPW - 2) // 2, step, 0)

    j = NPW - 2
    pltpu.make_async_copy(src(j), buf0, sl0).wait()
    pltpu.async_copy(buf0, dst(j), ss0)
    pltpu.make_async_copy(src(j + 1), buf1, sl1).wait()
    pltpu.async_copy(buf1, dst(j + 1), ss1)
    pltpu.make_async_copy(buf0, dst(j), ss0).wait()
    pltpu.make_async_copy(buf1, dst(j + 1), ss1).wait()


_patch_kernel = functools.partial(
    pl.kernel,
    out_type=jax.ShapeDtypeStruct((NP * PH, ROWBYTES), jnp.float32),
    mesh=plsc.VectorSubcoreMesh(
        core_axis_name="c", subcore_axis_name="s", num_cores=NC, num_subcores=NS
    ),
    scratch_types=[
        pltpu.VMEM((PH, ROWBYTES), jnp.float32),
        pltpu.VMEM((PH, ROWBYTES), jnp.float32),
        pltpu.SemaphoreType.DMA,
        pltpu.SemaphoreType.DMA,
        pltpu.SemaphoreType.DMA,
        pltpu.SemaphoreType.DMA,
    ],
)(_patch_body)


def kernel(x, mesh_pos, batch_idx):
    x2 = x.reshape(H, WCOLS)
    out = _patch_kernel(x2)
    return out.reshape(1, NP, PH, PW, C)
